# baseline (speedup 1.0000x reference)
import functools

import jax
import jax.numpy as jnp
from jax import lax
from jax.experimental import pallas as pl
from jax.experimental.pallas import tpu as pltpu
from jax.experimental.pallas import tpu_sc as plsc

_NC = 2
_NS = 16
_NW = _NC * _NS


def _sc_gather(table, ids2d, *, flat_base, rows, chunk):
    d = table.shape[1]
    s = ids2d.shape[1]
    per_w = rows // _NW
    nchunks = per_w // chunk
    mesh = plsc.VectorSubcoreMesh(core_axis_name="c", subcore_axis_name="s")

    @functools.partial(
        pl.kernel,
        out_type=jax.ShapeDtypeStruct((rows, d), jnp.float32),
        mesh=mesh,
        scratch_types=[
            pltpu.VMEM((chunk,), jnp.int32),
            pltpu.VMEM((chunk,), jnp.int32),
            pltpu.VMEM((chunk, d), jnp.float32),
            pltpu.VMEM((chunk, d), jnp.float32),
            pltpu.SemaphoreType.DMA,
            pltpu.SemaphoreType.DMA,
            pltpu.SemaphoreType.DMA,
            pltpu.SemaphoreType.DMA,
        ],
    )
    def gather_kernel(table_hbm, ids_hbm, out_hbm,
                      idx0, idx1, buf0, buf1, g0, g1, o0, o1):
        wid = lax.axis_index("s") * _NC + lax.axis_index("c")
        base = wid * per_w
        flat = flat_base + base
        row = flat // s
        col = flat % s
        idx = (idx0, idx1)
        buf = (buf0, buf1)
        gsem = (g0, g1)
        osem = (o0, o1)

        def start_gather(c, p):
            pltpu.sync_copy(
                ids_hbm.at[row, pl.ds(col + c * chunk, chunk)], idx[p])
            pltpu.async_copy(table_hbm.at[idx[p]], buf[p], gsem[p])

        def start_out(c, p):
            pltpu.async_copy(
                buf[p], out_hbm.at[pl.ds(base + c * chunk, chunk), :], osem[p])

        start_gather(0, 0)
        for c in range(1, nchunks):
            p = c % 2
            q = (c - 1) % 2
            if c >= 2:
                pltpu.make_async_copy(
                    buf[p], out_hbm.at[pl.ds(0, chunk), :], osem[p]).wait()
            start_gather(c, p)
            pltpu.make_async_copy(
                table_hbm.at[idx[q]], buf[q], gsem[q]).wait()
            start_out(c - 1, q)
        last = nchunks - 1
        pltpu.make_async_copy(
            table_hbm.at[idx[last % 2]], buf[last % 2], gsem[last % 2]).wait()
        start_out(last, last % 2)
        pltpu.make_async_copy(
            buf[0], out_hbm.at[pl.ds(0, chunk), :], osem[0]).wait()
        pltpu.make_async_copy(
            buf[1], out_hbm.at[pl.ds(0, chunk), :], osem[1]).wait()

    return gather_kernel(table, ids2d)


def _ln_body_aliased(buf_ref, seg_ref, g_ref, pos_ref, segtab_ref, gamma_ref,
                     beta_ref, o_ref):
    del buf_ref
    _ln_body(seg_ref, g_ref, pos_ref, segtab_ref, gamma_ref, beta_ref, o_ref)


def _ln_body(seg_ref, g_ref, pos_ref, segtab_ref, gamma_ref, beta_ref, o_ref):
    x = g_ref[...] + pos_ref[...]
    srow = seg_ref[0]
    w = (srow == 1).astype(jnp.float32).reshape(srow.shape[1], 1)
    s0 = segtab_ref[0:1, :]
    s1 = segtab_ref[1:2, :]
    x = x + s0 + w * (s1 - s0)
    mean = jnp.mean(x, axis=-1, keepdims=True)
    xc = x - mean
    var = jnp.mean(xc * xc, axis=-1, keepdims=True)
    y = xc * lax.rsqrt(var + 1e-12)
    o_ref[...] = y * gamma_ref[...] + beta_ref[...]


def _tc_ln(gathered, seg3d, pos_table, seg_table, gamma, beta, *, tile,
           total_rows, base_block, out_buf=None):
    rows, d = gathered.shape
    s = pos_table.shape[0]
    nseg = seg_table.shape[0]
    spt = s // tile
    nseq = rows // s
    grid = (spt, nseq)
    data_specs = [
        pl.BlockSpec((1, 1, tile), lambda j, i: (i, 0, j)),
        pl.BlockSpec((tile, d), lambda j, i: (i * spt + j, 0)),
        pl.BlockSpec((tile, d), lambda j, i: (j, 0)),
        pl.BlockSpec((nseg, d), lambda j, i: (0, 0)),
        pl.BlockSpec((1, d), lambda j, i: (0, 0)),
        pl.BlockSpec((1, d), lambda j, i: (0, 0)),
    ]
    out_spec = pl.BlockSpec((tile, d), lambda j, i: (base_block + i * spt + j, 0))
    out_shape = jax.ShapeDtypeStruct((total_rows, d), jnp.float32)
    args = (seg3d, gathered, pos_table, seg_table, gamma, beta)
    if out_buf is None:
        return pl.pallas_call(
            _ln_body, grid=grid, in_specs=data_specs, out_specs=out_spec,
            out_shape=out_shape,
        )(*args)
    return pl.pallas_call(
        _ln_body_aliased, grid=grid,
        in_specs=[pl.BlockSpec(memory_space=pl.ANY)] + data_specs,
        out_specs=out_spec, out_shape=out_shape,
        input_output_aliases={0: 0},
    )(out_buf, *args)


def kernel(input_ids, segment_ids, token_table, pos_table, seg_table, gamma, beta):
    b, s = input_ids.shape
    d = token_table.shape[1]
    bs = b * s
    seg3d = segment_ids.reshape(b, 1, s)
    gamma2 = gamma.reshape(1, -1)
    beta2 = beta.reshape(1, -1)
    nch = 8
    tc_tile = 512
    csz = bs // nch
    brows = b // nch
    out = None
    for i in range(nch):
        g = _sc_gather(token_table, input_ids,
                       flat_base=i * csz, rows=csz, chunk=64)
        out = _tc_ln(g, seg3d[i * brows:(i + 1) * brows], pos_table, seg_table,
                     gamma2, beta2, tile=tc_tile, total_rows=bs,
                     base_block=i * (csz // tc_tile), out_buf=out)
    return out.reshape(b, s, d)


# speedup vs baseline: 1.0300x; 1.0300x over previous
import functools

import jax
import jax.numpy as jnp
from jax import lax
from jax.experimental import pallas as pl
from jax.experimental.pallas import tpu as pltpu
from jax.experimental.pallas import tpu_sc as plsc

_NC = 2
_NS = 16
_NW = _NC * _NS


def _sc_gather(table, ids2d, *, flat_base, rows, chunk):
    d = table.shape[1]
    s = ids2d.shape[1]
    per_w = rows // _NW
    nchunks = per_w // chunk
    mesh = plsc.VectorSubcoreMesh(core_axis_name="c", subcore_axis_name="s")

    @functools.partial(
        pl.kernel,
        out_type=jax.ShapeDtypeStruct((rows, d), jnp.float32),
        mesh=mesh,
        scratch_types=[
            pltpu.VMEM((per_w,), jnp.int32),
            pltpu.VMEM((chunk, d), jnp.float32),
            pltpu.VMEM((chunk, d), jnp.float32),
            pltpu.SemaphoreType.DMA,
            pltpu.SemaphoreType.DMA,
            pltpu.SemaphoreType.DMA,
            pltpu.SemaphoreType.DMA,
        ],
    )
    def gather_kernel(table_hbm, ids_hbm, out_hbm,
                      idx_all, buf0, buf1, g0, g1, o0, o1):
        wid = lax.axis_index("s") * _NC + lax.axis_index("c")
        base = wid * per_w
        flat = flat_base + base
        row = flat // s
        col = flat % s
        buf = (buf0, buf1)
        gsem = (g0, g1)
        osem = (o0, o1)

        pltpu.sync_copy(ids_hbm.at[row, pl.ds(col, per_w)], idx_all)

        def start_gather(c, p):
            pltpu.async_copy(
                table_hbm.at[idx_all.at[pl.ds(c * chunk, chunk)]],
                buf[p], gsem[p])

        def wait_gather(c, p):
            pltpu.make_async_copy(
                table_hbm.at[idx_all.at[pl.ds(c * chunk, chunk)]],
                buf[p], gsem[p]).wait()

        def start_out(c, p):
            pltpu.async_copy(
                buf[p], out_hbm.at[pl.ds(base + c * chunk, chunk), :], osem[p])

        start_gather(0, 0)
        for c in range(1, nchunks):
            p = c % 2
            q = (c - 1) % 2
            if c >= 2:
                pltpu.make_async_copy(
                    buf[p], out_hbm.at[pl.ds(0, chunk), :], osem[p]).wait()
            start_gather(c, p)
            wait_gather(c - 1, q)
            start_out(c - 1, q)
        last = nchunks - 1
        wait_gather(last, last % 2)
        start_out(last, last % 2)
        pltpu.make_async_copy(
            buf[0], out_hbm.at[pl.ds(0, chunk), :], osem[0]).wait()
        pltpu.make_async_copy(
            buf[1], out_hbm.at[pl.ds(0, chunk), :], osem[1]).wait()

    return gather_kernel(table, ids2d)


def _ln_body_aliased(buf_ref, seg_ref, g_ref, pos_ref, segtab_ref, gamma_ref,
                     beta_ref, o_ref):
    del buf_ref
    _ln_body(seg_ref, g_ref, pos_ref, segtab_ref, gamma_ref, beta_ref, o_ref)


def _ln_body(seg_ref, g_ref, pos_ref, segtab_ref, gamma_ref, beta_ref, o_ref):
    x = g_ref[...] + pos_ref[...]
    srow = seg_ref[0]
    w = (srow == 1).astype(jnp.float32).reshape(srow.shape[1], 1)
    s0 = segtab_ref[0:1, :]
    s1 = segtab_ref[1:2, :]
    x = x + s0 + w * (s1 - s0)
    mean = jnp.mean(x, axis=-1, keepdims=True)
    xc = x - mean
    var = jnp.mean(xc * xc, axis=-1, keepdims=True)
    y = xc * lax.rsqrt(var + 1e-12)
    o_ref[...] = y * gamma_ref[...] + beta_ref[...]


def _tc_ln(gathered, seg3d, pos_table, seg_table, gamma, beta, *, tile,
           total_rows, base_block, out_buf=None):
    rows, d = gathered.shape
    s = pos_table.shape[0]
    nseg = seg_table.shape[0]
    spt = s // tile
    nseq = rows // s
    grid = (spt, nseq)
    data_specs = [
        pl.BlockSpec((1, 1, tile), lambda j, i: (i, 0, j)),
        pl.BlockSpec((tile, d), lambda j, i: (i * spt + j, 0)),
        pl.BlockSpec((tile, d), lambda j, i: (j, 0)),
        pl.BlockSpec((nseg, d), lambda j, i: (0, 0)),
        pl.BlockSpec((1, d), lambda j, i: (0, 0)),
        pl.BlockSpec((1, d), lambda j, i: (0, 0)),
    ]
    out_spec = pl.BlockSpec((tile, d), lambda j, i: (base_block + i * spt + j, 0))
    out_shape = jax.ShapeDtypeStruct((total_rows, d), jnp.float32)
    args = (seg3d, gathered, pos_table, seg_table, gamma, beta)
    if out_buf is None:
        return pl.pallas_call(
            _ln_body, grid=grid, in_specs=data_specs, out_specs=out_spec,
            out_shape=out_shape,
        )(*args)
    return pl.pallas_call(
        _ln_body_aliased, grid=grid,
        in_specs=[pl.BlockSpec(memory_space=pl.ANY)] + data_specs,
        out_specs=out_spec, out_shape=out_shape,
        input_output_aliases={0: 0},
    )(out_buf, *args)


def kernel(input_ids, segment_ids, token_table, pos_table, seg_table, gamma, beta):
    b, s = input_ids.shape
    d = token_table.shape[1]
    bs = b * s
    seg3d = segment_ids.reshape(b, 1, s)
    gamma2 = gamma.reshape(1, -1)
    beta2 = beta.reshape(1, -1)
    nch = 4
    tc_tile = 512
    csz = bs // nch
    brows = b // nch
    out = None
    for i in range(nch):
        g = _sc_gather(token_table, input_ids,
                       flat_base=i * csz, rows=csz, chunk=64)
        out = _tc_ln(g, seg3d[i * brows:(i + 1) * brows], pos_table, seg_table,
                     gamma2, beta2, tile=tc_tile, total_rows=bs,
                     base_block=i * (csz // tc_tile), out_buf=out)
    return out.reshape(b, s, d)


# speedup vs baseline: 1.0517x; 1.0210x over previous
import functools

import jax
import jax.numpy as jnp
from jax import lax
from jax.experimental import pallas as pl
from jax.experimental.pallas import tpu as pltpu
from jax.experimental.pallas import tpu_sc as plsc

_NC = 2
_NS = 16
_NW = _NC * _NS


def _sc_gather(table, ids2d, *, flat_base, rows, chunk):
    d = table.shape[1]
    s = ids2d.shape[1]
    per_w = rows // _NW
    nchunks = per_w // chunk
    mesh = plsc.VectorSubcoreMesh(core_axis_name="c", subcore_axis_name="s")

    @functools.partial(
        pl.kernel,
        out_type=jax.ShapeDtypeStruct((rows, d), jnp.float32),
        mesh=mesh,
        scratch_types=[
            pltpu.VMEM((per_w,), jnp.int32),
            pltpu.VMEM((chunk, d), jnp.float32),
            pltpu.VMEM((chunk, d), jnp.float32),
            pltpu.SemaphoreType.DMA,
            pltpu.SemaphoreType.DMA,
            pltpu.SemaphoreType.DMA,
            pltpu.SemaphoreType.DMA,
        ],
    )
    def gather_kernel(table_hbm, ids_hbm, out_hbm,
                      idx_all, buf0, buf1, g0, g1, o0, o1):
        wid = lax.axis_index("s") * _NC + lax.axis_index("c")
        base = wid * per_w
        flat = flat_base + base
        row = flat // s
        col = flat % s
        buf = (buf0, buf1)
        gsem = (g0, g1)
        osem = (o0, o1)

        pltpu.sync_copy(ids_hbm.at[row, pl.ds(col, per_w)], idx_all)

        def start_gather(c, p):
            pltpu.async_copy(
                table_hbm.at[idx_all.at[pl.ds(c * chunk, chunk)]],
                buf[p], gsem[p])

        def wait_gather(c, p):
            pltpu.make_async_copy(
                table_hbm.at[idx_all.at[pl.ds(c * chunk, chunk)]],
                buf[p], gsem[p]).wait()

        def start_out(c, p):
            pltpu.async_copy(
                buf[p], out_hbm.at[pl.ds(base + c * chunk, chunk), :], osem[p])

        start_gather(0, 0)
        for c in range(1, nchunks):
            p = c % 2
            q = (c - 1) % 2
            if c >= 2:
                pltpu.make_async_copy(
                    buf[p], out_hbm.at[pl.ds(0, chunk), :], osem[p]).wait()
            start_gather(c, p)
            wait_gather(c - 1, q)
            start_out(c - 1, q)
        last = nchunks - 1
        wait_gather(last, last % 2)
        start_out(last, last % 2)
        pltpu.make_async_copy(
            buf[0], out_hbm.at[pl.ds(0, chunk), :], osem[0]).wait()
        pltpu.make_async_copy(
            buf[1], out_hbm.at[pl.ds(0, chunk), :], osem[1]).wait()

    return gather_kernel(table, ids2d)


def _ln_body_aliased(buf_ref, seg_ref, g_ref, pos_ref, segtab_ref, gamma_ref,
                     beta_ref, o_ref):
    del buf_ref
    _ln_body(seg_ref, g_ref, pos_ref, segtab_ref, gamma_ref, beta_ref, o_ref)


def _ln_body(seg_ref, g_ref, pos_ref, segtab_ref, gamma_ref, beta_ref, o_ref):
    m, _, s = seg_ref.shape
    d = g_ref.shape[1]
    x = g_ref[...].reshape(m, s, d) + pos_ref[...][None]
    srow = seg_ref[:, 0, :]
    w = (srow == 1).astype(jnp.float32).reshape(m, s, 1)
    s0 = segtab_ref[0:1, :]
    s1 = segtab_ref[1:2, :]
    x = x + s0 + w * (s1 - s0)
    mean = jnp.mean(x, axis=-1, keepdims=True)
    xc = x - mean
    var = jnp.mean(xc * xc, axis=-1, keepdims=True)
    y = xc * lax.rsqrt(var + 1e-12)
    o_ref[...] = (y * gamma_ref[...] + beta_ref[...]).reshape(m * s, d)


def _tc_ln(gathered, seg3d, pos_table, seg_table, gamma, beta, *, tile,
           total_rows, base_block, out_buf=None):
    rows, d = gathered.shape
    s = pos_table.shape[0]
    nseg = seg_table.shape[0]
    m = tile // s
    grid = (rows // tile,)
    data_specs = [
        pl.BlockSpec((m, 1, s), lambda i: (i, 0, 0)),
        pl.BlockSpec((tile, d), lambda i: (i, 0)),
        pl.BlockSpec((s, d), lambda i: (0, 0)),
        pl.BlockSpec((nseg, d), lambda i: (0, 0)),
        pl.BlockSpec((1, d), lambda i: (0, 0)),
        pl.BlockSpec((1, d), lambda i: (0, 0)),
    ]
    out_spec = pl.BlockSpec((tile, d), lambda i: (base_block + i, 0))
    out_shape = jax.ShapeDtypeStruct((total_rows, d), jnp.float32)
    args = (seg3d, gathered, pos_table, seg_table, gamma, beta)
    if out_buf is None:
        return pl.pallas_call(
            _ln_body, grid=grid, in_specs=data_specs, out_specs=out_spec,
            out_shape=out_shape,
        )(*args)
    return pl.pallas_call(
        _ln_body_aliased, grid=grid,
        in_specs=[pl.BlockSpec(memory_space=pl.ANY)] + data_specs,
        out_specs=out_spec, out_shape=out_shape,
        input_output_aliases={0: 0},
    )(out_buf, *args)


def kernel(input_ids, segment_ids, token_table, pos_table, seg_table, gamma, beta):
    b, s = input_ids.shape
    d = token_table.shape[1]
    bs = b * s
    seg3d = segment_ids.reshape(b, 1, s)
    gamma2 = gamma.reshape(1, -1)
    beta2 = beta.reshape(1, -1)
    nch = 4
    tc_tile = 1024
    csz = bs // nch
    brows = b // nch
    out = None
    for i in range(nch):
        g = _sc_gather(token_table, input_ids,
                       flat_base=i * csz, rows=csz, chunk=64)
        out = _tc_ln(g, seg3d[i * brows:(i + 1) * brows], pos_table, seg_table,
                     gamma2, beta2, tile=tc_tile, total_rows=bs,
                     base_block=i * (csz // tc_tile), out_buf=out)
    return out.reshape(b, s, d)


# speedup vs baseline: 1.0644x; 1.0121x over previous
import functools

import jax
import jax.numpy as jnp
from jax import lax
from jax.experimental import pallas as pl
from jax.experimental.pallas import tpu as pltpu
from jax.experimental.pallas import tpu_sc as plsc

_NC = 2
_NS = 16
_NW = _NC * _NS


def _sc_gather(table, ids2d, *, flat_base, rows, chunk):
    d = table.shape[1]
    s = ids2d.shape[1]
    per_w = rows // _NW
    nchunks = per_w // chunk
    mesh = plsc.VectorSubcoreMesh(core_axis_name="c", subcore_axis_name="s")

    @functools.partial(
        pl.kernel,
        out_type=jax.ShapeDtypeStruct((rows, d), jnp.float32),
        mesh=mesh,
        scratch_types=[
            pltpu.VMEM((per_w,), jnp.int32),
            pltpu.VMEM((chunk, d), jnp.float32),
            pltpu.VMEM((chunk, d), jnp.float32),
            pltpu.SemaphoreType.DMA,
            pltpu.SemaphoreType.DMA,
            pltpu.SemaphoreType.DMA,
            pltpu.SemaphoreType.DMA,
        ],
    )
    def gather_kernel(table_hbm, ids_hbm, out_hbm,
                      idx_all, buf0, buf1, g0, g1, o0, o1):
        wid = lax.axis_index("s") * _NC + lax.axis_index("c")
        base = wid * per_w
        flat = flat_base + base
        row = flat // s
        col = flat % s
        buf = (buf0, buf1)
        gsem = (g0, g1)
        osem = (o0, o1)

        pltpu.sync_copy(ids_hbm.at[row, pl.ds(col, per_w)], idx_all)

        def start_gather(c, p):
            pltpu.async_copy(
                table_hbm.at[idx_all.at[pl.ds(c * chunk, chunk)]],
                buf[p], gsem[p])

        def wait_gather(c, p):
            pltpu.make_async_copy(
                table_hbm.at[idx_all.at[pl.ds(c * chunk, chunk)]],
                buf[p], gsem[p]).wait()

        def start_out(c, p):
            pltpu.async_copy(
                buf[p], out_hbm.at[pl.ds(base + c * chunk, chunk), :], osem[p])

        start_gather(0, 0)
        for c in range(1, nchunks):
            p = c % 2
            q = (c - 1) % 2
            if c >= 2:
                pltpu.make_async_copy(
                    buf[p], out_hbm.at[pl.ds(0, chunk), :], osem[p]).wait()
            start_gather(c, p)
            wait_gather(c - 1, q)
            start_out(c - 1, q)
        last = nchunks - 1
        wait_gather(last, last % 2)
        start_out(last, last % 2)
        pltpu.make_async_copy(
            buf[0], out_hbm.at[pl.ds(0, chunk), :], osem[0]).wait()
        pltpu.make_async_copy(
            buf[1], out_hbm.at[pl.ds(0, chunk), :], osem[1]).wait()

    return gather_kernel(table, ids2d)


def _ln_body_aliased(buf_ref, seg_ref, g_ref, pos_ref, segtab_ref, gamma_ref,
                     beta_ref, o_ref):
    del buf_ref
    _ln_body(seg_ref, g_ref, pos_ref, segtab_ref, gamma_ref, beta_ref, o_ref)


def _ln_body(seg_ref, g_ref, pos_ref, segtab_ref, gamma_ref, beta_ref, o_ref):
    m, _, s = seg_ref.shape
    d = g_ref.shape[1]
    x = g_ref[...].reshape(m, s, d) + pos_ref[...][None]
    srow = seg_ref[:, 0, :]
    w = (srow == 1).astype(jnp.float32).reshape(m, s, 1)
    s0 = segtab_ref[0:1, :]
    s1 = segtab_ref[1:2, :]
    x = x + s0 + w * (s1 - s0)
    mean = jnp.mean(x, axis=-1, keepdims=True)
    xc = x - mean
    var = jnp.mean(xc * xc, axis=-1, keepdims=True)
    y = xc * lax.rsqrt(var + 1e-12)
    o_ref[...] = (y * gamma_ref[...] + beta_ref[...]).reshape(m * s, d)


def _tc_ln(gathered, seg3d, pos_table, seg_table, gamma, beta, *, tile,
           total_rows, base_block, out_buf=None):
    rows, d = gathered.shape
    s = pos_table.shape[0]
    nseg = seg_table.shape[0]
    m = tile // s
    grid = (rows // tile,)
    data_specs = [
        pl.BlockSpec((m, 1, s), lambda i: (i, 0, 0)),
        pl.BlockSpec((tile, d), lambda i: (i, 0)),
        pl.BlockSpec((s, d), lambda i: (0, 0)),
        pl.BlockSpec((nseg, d), lambda i: (0, 0)),
        pl.BlockSpec((1, d), lambda i: (0, 0)),
        pl.BlockSpec((1, d), lambda i: (0, 0)),
    ]
    out_spec = pl.BlockSpec((tile, d), lambda i: (base_block + i, 0))
    out_shape = jax.ShapeDtypeStruct((total_rows, d), jnp.float32)
    args = (seg3d, gathered, pos_table, seg_table, gamma, beta)
    if out_buf is None:
        return pl.pallas_call(
            _ln_body, grid=grid, in_specs=data_specs, out_specs=out_spec,
            out_shape=out_shape,
        )(*args)
    return pl.pallas_call(
        _ln_body_aliased, grid=grid,
        in_specs=[pl.BlockSpec(memory_space=pl.ANY)] + data_specs,
        out_specs=out_spec, out_shape=out_shape,
        input_output_aliases={0: 0},
    )(out_buf, *args)


def kernel(input_ids, segment_ids, token_table, pos_table, seg_table, gamma, beta):
    b, s = input_ids.shape
    d = token_table.shape[1]
    bs = b * s
    seg3d = segment_ids.reshape(b, 1, s)
    gamma2 = gamma.reshape(1, -1)
    beta2 = beta.reshape(1, -1)
    nch = 4
    tc_tile = 2048
    csz = bs // nch
    brows = b // nch
    out = None
    for i in range(nch):
        g = _sc_gather(token_table, input_ids,
                       flat_base=i * csz, rows=csz, chunk=64)
        out = _tc_ln(g, seg3d[i * brows:(i + 1) * brows], pos_table, seg_table,
                     gamma2, beta2, tile=tc_tile, total_rows=bs,
                     base_block=i * (csz // tc_tile), out_buf=out)
    return out.reshape(b, s, d)
